# Initial kernel scaffold; baseline (speedup 1.0000x reference)
import jax
import jax.numpy as jnp
from jax.experimental import pallas as pl


def kernel(x, y, pos, edge_index, W_unary, g_unary, b_unary, W_pair, g_pair, b_pair, c, W_mlp, g_mlp, b_mlp, W_fuse, g_fuse, b_fuse):
    raise NotImplementedError("write your pallas kernel here")


# speedup vs baseline: 9.2060x; 9.2060x over previous
import functools

import jax
import jax.numpy as jnp
from jax import lax
from jax.experimental import pallas as pl
from jax.experimental.pallas import tpu as pltpu
from jax.experimental.pallas import tpu_sc as plsc

NC = 2
NS = 16
NWK = NC * NS
CH = 128

f32 = jnp.float32
i32 = jnp.int32


def _cdiv(a, b):
    return (a + b - 1) // b


def _prep_w_body(c_ref, bc_ref, cb_ref):
    cc = c_ref[:]
    C = lax.dot_general(cc, cc, (((0,), (0,)), ((), ())),
                        preferred_element_type=f32)
    h = C.shape[0]
    ii = lax.broadcasted_iota(i32, (h, h), 0)
    jj = lax.broadcasted_iota(i32, (h, h), 1)
    I = jnp.where(ii == jj, 1.0, 0.0).astype(f32)
    A = I + C
    X = 0.5 * I
    for _ in range(12):
        AX = jnp.dot(A, X, preferred_element_type=f32)
        X = jnp.dot(X, 2.0 * I - AX, preferred_element_type=f32)
    bc_ref[:] = X
    cb_ref[:] = jnp.dot(C, X, preferred_element_type=f32)


def _pre_body(x_ref, y_ref, wu_ref, wp_ref, p1_ref, p2_ref, s_ref):
    p1 = jnp.dot(x_ref[:], wu_ref[:], preferred_element_type=f32)
    p2 = jnp.dot(y_ref[:], wp_ref[:], preferred_element_type=f32)
    p1_ref[:] = p1
    p2_ref[:] = p2
    s_ref[0, 0, 0, :] = jnp.sum(p1, axis=0)
    s_ref[0, 0, 1, :] = jnp.sum(p1 * p1, axis=0)
    s_ref[0, 0, 2, :] = jnp.sum(p2, axis=0)
    s_ref[0, 0, 3, :] = jnp.sum(p2 * p2, axis=0)


def _norm_body(nreal, p1_ref, p2_ref, s_ref, gu_ref, bu_ref, gp_ref, bp_ref,
               bc_ref, sn_ref, zb_ref, h0_ref):
    st = jnp.sum(s_ref[:, 0, :, :], axis=0)
    nf = float(nreal)
    m1 = st[0] / nf
    v1 = st[1] / nf - m1 * m1
    m2 = st[2] / nf
    v2 = st[3] / nf - m2 * m2
    xh = gu_ref[:] * (p1_ref[:] - m1) * lax.rsqrt(v1 + 1e-5) + bu_ref[:]
    sn = gp_ref[:] * (p2_ref[:] - m2) * lax.rsqrt(v2 + 1e-5) + bp_ref[:]
    sn_ref[:] = sn
    h0_ref[:] = xh
    zb_ref[:] = jnp.dot(xh, bc_ref[:], preferred_element_type=f32)


def _update_body(nreal, rrows, up0_ref, up1_ref, den0_ref, den1_ref, zb_ref,
                 cb_ref, h_ref):
    U = up0_ref[:] + up1_ref[:]
    den = den0_ref[:] + den1_ref[:]
    safe = jnp.where(den > 0, den, 1.0)
    agg = jnp.where(den > 0, U / safe, 0.0)
    h = zb_ref[:] + jnp.dot(agg, cb_ref[:], preferred_element_type=f32)
    rid = pl.program_id(0) * rrows + lax.broadcasted_iota(i32, (rrows, 1), 0)
    h_ref[:] = jnp.where(rid < nreal, h, 0.0)


def _posta_body(h_ref, wm_ref, g1_ref, s_ref):
    g1 = jnp.dot(h_ref[:], wm_ref[:], preferred_element_type=f32)
    g1_ref[:] = g1
    s_ref[0, 0, 0, :] = jnp.sum(g1, axis=0)
    s_ref[0, 0, 1, :] = jnp.sum(g1 * g1, axis=0)


def _postb_body(nreal, rrows, g1_ref, s_ref, gm_ref, bm_ref, y_ref, wfh_ref,
                wfy_ref, g2_ref, s2_ref):
    st = jnp.sum(s_ref[:, 0, :, :], axis=0)
    nf = float(nreal)
    m = st[0] / nf
    v = st[1] / nf - m * m
    hm = gm_ref[:] * (g1_ref[:] - m) * lax.rsqrt(v + 1e-5) + bm_ref[:]
    hm = jnp.where(hm >= 0, hm, 0.01 * hm)
    rid = pl.program_id(0) * rrows + lax.broadcasted_iota(i32, (rrows, 1), 0)
    hm = jnp.where(rid < nreal, hm, 0.0)
    g2 = (jnp.dot(hm, wfh_ref[:], preferred_element_type=f32)
          + jnp.dot(y_ref[:], wfy_ref[:], preferred_element_type=f32))
    g2_ref[:] = g2
    s2_ref[0, 0, 0, :] = jnp.sum(g2, axis=0)
    s2_ref[0, 0, 1, :] = jnp.sum(g2 * g2, axis=0)


def _postc_body(nreal, g2_ref, s_ref, gf_ref, bf_ref, out_ref):
    st = jnp.sum(s_ref[:, 0, :, :], axis=0)
    nf = float(nreal)
    m = st[0] / nf
    v = st[1] / nf - m * m
    o = gf_ref[:] * (g2_ref[:] - m) * lax.rsqrt(v + 1e-5) + bf_ref[:]
    out_ref[:] = jnp.where(o >= 0, o, 0.01 * o)


def _sc_pass1_body(nch, hdim, s_ref, i_ref, j_ref, ex_ref, den0_ref, den1_ref,
                   ivec, jvec, exbuf, bi, bj, zbuf, den_sh):
    c_ax = lax.axis_index("c")
    s_ax = lax.axis_index("s")
    wid = s_ax * NC + c_ax
    tslice = den_sh.shape[0] // NS
    off = s_ax * tslice

    def zb_body(k, carry):
        zbuf[pl.ds(k * 16, 16)] = jnp.zeros((16,), f32)
        return carry
    lax.fori_loop(0, tslice // 16, zb_body, 0)
    pltpu.sync_copy(zbuf, den_sh.at[pl.ds(off, tslice)])
    plsc.subcore_barrier()

    base = nch // NWK
    rem = nch % NWK
    cnt = base + jnp.where(wid < rem, 1, 0)
    start = wid * base + jnp.minimum(wid, rem)
    lane = lax.iota(i32, 16)

    nh = hdim // 16

    def chunk(ci, carry):
        pltpu.sync_copy(i_ref.at[ci], ivec)
        pltpu.sync_copy(j_ref.at[ci], jvec)
        pltpu.sync_copy(s_ref.at[ivec], bi)
        pltpu.sync_copy(s_ref.at[jvec], bj)

        for g in range(CH // 16):
            acc = jnp.zeros((16,), f32)
            for l in range(16):
                r = g * 16 + l
                row = jnp.zeros((16,), f32)
                for hh in range(nh):
                    sl = pl.ds(hh * 16, 16)
                    d = bi[r, sl] - bj[r, sl]
                    row = row + d * d
                row = row + lax.rev(row, (0,))
                sd = row[0]
                for l2 in range(1, 8):
                    sd = sd + row[l2]
                acc = jnp.where(lane == l, sd, acc)
            exbuf[pl.ds(g * 16, 16)] = jnp.exp(64.0 - acc)

        pltpu.sync_copy(exbuf, ex_ref.at[ci])
        pltpu.sync_copy(exbuf, den_sh.at[ivec], add=True)
        return carry
    lax.fori_loop(start, start + cnt, chunk, 0)

    plsc.subcore_barrier()
    pltpu.sync_copy(den_sh.at[pl.ds(off, tslice)], zbuf)

    @pl.when(c_ax == 0)
    def _():
        pltpu.sync_copy(zbuf, den0_ref.at[pl.ds(off, tslice)])

    @pl.when(c_ax == 1)
    def _():
        pltpu.sync_copy(zbuf, den1_ref.at[pl.ds(off, tslice)])


def _sc_pass2_body(nch, hdim, h_ref, ex_ref, i_ref, j_ref, up0_ref, up1_ref,
                   ivec, jvec, exv, rows, zrows, U_sh):
    c_ax = lax.axis_index("c")
    s_ax = lax.axis_index("s")
    wid = s_ax * NC + c_ax
    tslice = U_sh.shape[0] // NS
    off = s_ax * tslice
    nh = hdim // 16

    for k in range(zrows.shape[0]):
        for hh in range(nh):
            zrows[k, pl.ds(hh * 16, 16)] = jnp.zeros((16,), f32)

    def zc_body(k, carry):
        pltpu.sync_copy(zrows, U_sh.at[pl.ds(off + k * 64, 64)])
        return carry
    lax.fori_loop(0, tslice // 64, zc_body, 0)
    plsc.subcore_barrier()

    base = nch // NWK
    rem = nch % NWK
    cnt = base + jnp.where(wid < rem, 1, 0)
    start = wid * base + jnp.minimum(wid, rem)

    def chunk(ci, carry):
        pltpu.sync_copy(i_ref.at[ci], ivec)
        pltpu.sync_copy(j_ref.at[ci], jvec)
        pltpu.sync_copy(ex_ref.at[ci], exv)
        pltpu.sync_copy(h_ref.at[jvec], rows)

        for g in range(CH // 16):
            evec = exv[pl.ds(g * 16, 16)]
            for l in range(16):
                se = evec[l]
                r = g * 16 + l
                for hh in range(nh):
                    sl = pl.ds(hh * 16, 16)
                    rows[r, sl] = rows[r, sl] * se

        pltpu.sync_copy(rows, U_sh.at[ivec], add=True)
        return carry
    lax.fori_loop(start, start + cnt, chunk, 0)

    plsc.subcore_barrier()

    def dr_body(k, carry):
        pltpu.sync_copy(U_sh.at[pl.ds(off + k * 64, 64)], zrows)

        @pl.when(c_ax == 0)
        def _():
            pltpu.sync_copy(zrows, up0_ref.at[pl.ds(off + k * 64, 64)])

        @pl.when(c_ax == 1)
        def _():
            pltpu.sync_copy(zrows, up1_ref.at[pl.ds(off + k * 64, 64)])
        return carry
    lax.fori_loop(0, tslice // 64, dr_body, 0)


def kernel(x, y, pos, edge_index, W_unary, g_unary, b_unary, W_pair, g_pair,
           b_pair, c, W_mlp, g_mlp, b_mlp, W_fuse, g_fuse, b_fuse):
    n, d = x.shape
    e = edge_index.shape[1]
    h = W_unary.shape[1]
    out_d = W_mlp.shape[1]
    assert e % CH == 0
    np2 = _cdiv(n, 1024) * 1024
    rrows = 1024
    nb = np2 // rrows
    nch = e // CH
    SDS = jax.ShapeDtypeStruct

    xp = jnp.pad(x, ((0, np2 - n), (0, 0)))
    yp = jnp.pad(y, ((0, np2 - n), (0, 0)))
    i2 = edge_index[0].reshape(nch, CH)
    j2 = edge_index[1].reshape(nch, CH)
    wf_h = W_fuse[:out_d]
    wf_y = W_fuse[out_d:]

    bc, cb = pl.pallas_call(
        _prep_w_body,
        out_shape=[SDS((h, h), f32), SDS((h, h), f32)],
    )(c)

    rowblk = lambda bdim: pl.BlockSpec((rrows, bdim), lambda b: (b, 0))
    full2 = lambda s0, s1: pl.BlockSpec((s0, s1), lambda b: (0, 0))
    full1 = lambda s0: pl.BlockSpec((s0,), lambda b: (0,))
    statblk = lambda k: pl.BlockSpec((1, 1, k, h), lambda b: (b, 0, 0, 0))
    statall = lambda k: pl.BlockSpec((nb, 1, k, h), lambda b: (0, 0, 0, 0))
    statblk_o = lambda k: pl.BlockSpec((1, 1, k, out_d), lambda b: (b, 0, 0, 0))
    statall_o = lambda k: pl.BlockSpec((nb, 1, k, out_d), lambda b: (0, 0, 0, 0))

    p1, p2, s1 = pl.pallas_call(
        _pre_body,
        grid=(nb,),
        in_specs=[rowblk(d), rowblk(d), full2(d, h), full2(d, h)],
        out_specs=[rowblk(h), rowblk(h), statblk(4)],
        out_shape=[SDS((np2, h), f32), SDS((np2, h), f32),
                   SDS((nb, 1, 4, h), f32)],
    )(xp, yp, W_unary, W_pair)

    s_nodes, zb, h0 = pl.pallas_call(
        functools.partial(_norm_body, n),
        grid=(nb,),
        in_specs=[rowblk(h), rowblk(h), statall(4), full1(h), full1(h),
                  full1(h), full1(h), full2(h, h)],
        out_specs=[rowblk(h), rowblk(h), rowblk(h)],
        out_shape=[SDS((np2, h), f32), SDS((np2, h), f32), SDS((np2, h), f32)],
    )(p1, p2, s1, g_unary, b_unary, g_pair, b_pair, bc)

    mesh = plsc.VectorSubcoreMesh(core_axis_name="c", subcore_axis_name="s",
                                  num_cores=NC, num_subcores=NS)
    ex2, den0, den1 = pl.kernel(
        functools.partial(_sc_pass1_body, nch, h),
        out_type=[SDS((nch, CH), f32), SDS((np2,), f32), SDS((np2,), f32)],
        mesh=mesh,
        compiler_params=pltpu.CompilerParams(use_tc_tiling_on_sc=False),
        scratch_types=[
            pltpu.VMEM((CH,), i32),
            pltpu.VMEM((CH,), i32),
            pltpu.VMEM((CH,), f32),
            pltpu.VMEM((CH, h), f32),
            pltpu.VMEM((CH, h), f32),
            pltpu.VMEM((np2 // NS,), f32),
            pltpu.VMEM_SHARED((np2,), f32),
        ],
    )(s_nodes, i2, j2)
    den0 = den0.reshape(np2, 1)
    den1 = den1.reshape(np2, 1)

    hcur = h0
    for _ in range(2):
        up0, up1 = pl.kernel(
            functools.partial(_sc_pass2_body, nch, h),
            out_type=[SDS((np2, h), f32), SDS((np2, h), f32)],
            mesh=mesh,
            compiler_params=pltpu.CompilerParams(use_tc_tiling_on_sc=False),
            scratch_types=[
                pltpu.VMEM((CH,), i32),
                pltpu.VMEM((CH,), i32),
                pltpu.VMEM((CH,), f32),
                pltpu.VMEM((CH, h), f32),
                pltpu.VMEM((64, h), f32),
                pltpu.VMEM_SHARED((np2, h), f32),
            ],
        )(hcur, ex2, i2, j2)

        colblk = pl.BlockSpec((rrows, 1), lambda b: (b, 0))
        hcur = pl.pallas_call(
            functools.partial(_update_body, n, rrows),
            grid=(nb,),
            in_specs=[rowblk(h), rowblk(h), colblk, colblk,
                      rowblk(h), full2(h, h)],
            out_specs=rowblk(h),
            out_shape=SDS((np2, h), f32),
        )(up0, up1, den0, den1, zb, cb)

    g1, s2 = pl.pallas_call(
        _posta_body,
        grid=(nb,),
        in_specs=[rowblk(h), full2(h, out_d)],
        out_specs=[rowblk(out_d), statblk_o(2)],
        out_shape=[SDS((np2, out_d), f32), SDS((nb, 1, 2, out_d), f32)],
    )(hcur, W_mlp)

    g2, s3 = pl.pallas_call(
        functools.partial(_postb_body, n, rrows),
        grid=(nb,),
        in_specs=[rowblk(out_d), statall_o(2), full1(out_d), full1(out_d),
                  rowblk(d), full2(out_d, out_d), full2(d, out_d)],
        out_specs=[rowblk(out_d), statblk_o(2)],
        out_shape=[SDS((np2, out_d), f32), SDS((nb, 1, 2, out_d), f32)],
    )(g1, s2, g_mlp, b_mlp, yp, wf_h, wf_y)

    outp = pl.pallas_call(
        functools.partial(_postc_body, n),
        grid=(nb,),
        in_specs=[rowblk(out_d), statall_o(2), full1(out_d), full1(out_d)],
        out_specs=rowblk(out_d),
        out_shape=SDS((np2, out_d), f32),
    )(g2, s3, g_fuse, b_fuse)

    return outp[:n]


# speedup vs baseline: 13.0410x; 1.4166x over previous
import functools

import jax
import jax.numpy as jnp
from jax import lax
from jax.experimental import pallas as pl
from jax.experimental.pallas import tpu as pltpu
from jax.experimental.pallas import tpu_sc as plsc

NC = 2
NS = 16
NWK = NC * NS
CH = 128

f32 = jnp.float32
i32 = jnp.int32


def _cdiv(a, b):
    return (a + b - 1) // b


def _prep_w_body(c_ref, bc_ref, cb_ref):
    cc = c_ref[:]
    C = lax.dot_general(cc, cc, (((0,), (0,)), ((), ())),
                        preferred_element_type=f32)
    h = C.shape[0]
    ii = lax.broadcasted_iota(i32, (h, h), 0)
    jj = lax.broadcasted_iota(i32, (h, h), 1)
    I = jnp.where(ii == jj, 1.0, 0.0).astype(f32)
    A = I + C
    X = 0.5 * I
    for _ in range(12):
        AX = jnp.dot(A, X, preferred_element_type=f32)
        X = jnp.dot(X, 2.0 * I - AX, preferred_element_type=f32)
    bc_ref[:] = X
    cb_ref[:] = jnp.dot(C, X, preferred_element_type=f32)


def _pre_body(x_ref, y_ref, wu_ref, wp_ref, p1_ref, p2_ref, s_ref):
    p1 = jnp.dot(x_ref[:], wu_ref[:], preferred_element_type=f32)
    p2 = jnp.dot(y_ref[:], wp_ref[:], preferred_element_type=f32)
    p1_ref[:] = p1
    p2_ref[:] = p2
    s_ref[0, 0, 0, :] = jnp.sum(p1, axis=0)
    s_ref[0, 0, 1, :] = jnp.sum(p1 * p1, axis=0)
    s_ref[0, 0, 2, :] = jnp.sum(p2, axis=0)
    s_ref[0, 0, 3, :] = jnp.sum(p2 * p2, axis=0)


def _norm_body(nreal, p1_ref, p2_ref, s_ref, gu_ref, bu_ref, gp_ref, bp_ref,
               bc_ref, sn_ref, zb_ref, h0_ref):
    st = jnp.sum(s_ref[:, 0, :, :], axis=0)
    nf = float(nreal)
    m1 = st[0] / nf
    v1 = st[1] / nf - m1 * m1
    m2 = st[2] / nf
    v2 = st[3] / nf - m2 * m2
    xh = gu_ref[:] * (p1_ref[:] - m1) * lax.rsqrt(v1 + 1e-5) + bu_ref[:]
    sn = gp_ref[:] * (p2_ref[:] - m2) * lax.rsqrt(v2 + 1e-5) + bp_ref[:]
    sn_ref[:] = sn
    h0_ref[:] = xh
    zb_ref[:] = jnp.dot(xh, bc_ref[:], preferred_element_type=f32)


def _update_body(nreal, rrows, up0_ref, up1_ref, den0_ref, den1_ref, zb_ref,
                 cb_ref, h_ref):
    U = up0_ref[:] + up1_ref[:]
    den = den0_ref[:] + den1_ref[:]
    safe = jnp.where(den > 0, den, 1.0)
    agg = jnp.where(den > 0, U / safe, 0.0)
    h = zb_ref[:] + jnp.dot(agg, cb_ref[:], preferred_element_type=f32)
    rid = pl.program_id(0) * rrows + lax.broadcasted_iota(i32, (rrows, 1), 0)
    h_ref[:] = jnp.where(rid < nreal, h, 0.0)


def _posta_body(h_ref, wm_ref, g1_ref, s_ref):
    g1 = jnp.dot(h_ref[:], wm_ref[:], preferred_element_type=f32)
    g1_ref[:] = g1
    s_ref[0, 0, 0, :] = jnp.sum(g1, axis=0)
    s_ref[0, 0, 1, :] = jnp.sum(g1 * g1, axis=0)


def _postb_body(nreal, rrows, g1_ref, s_ref, gm_ref, bm_ref, y_ref, wfh_ref,
                wfy_ref, g2_ref, s2_ref):
    st = jnp.sum(s_ref[:, 0, :, :], axis=0)
    nf = float(nreal)
    m = st[0] / nf
    v = st[1] / nf - m * m
    hm = gm_ref[:] * (g1_ref[:] - m) * lax.rsqrt(v + 1e-5) + bm_ref[:]
    hm = jnp.where(hm >= 0, hm, 0.01 * hm)
    rid = pl.program_id(0) * rrows + lax.broadcasted_iota(i32, (rrows, 1), 0)
    hm = jnp.where(rid < nreal, hm, 0.0)
    g2 = (jnp.dot(hm, wfh_ref[:], preferred_element_type=f32)
          + jnp.dot(y_ref[:], wfy_ref[:], preferred_element_type=f32))
    g2_ref[:] = g2
    s2_ref[0, 0, 0, :] = jnp.sum(g2, axis=0)
    s2_ref[0, 0, 1, :] = jnp.sum(g2 * g2, axis=0)


def _postc_body(nreal, g2_ref, s_ref, gf_ref, bf_ref, out_ref):
    st = jnp.sum(s_ref[:, 0, :, :], axis=0)
    nf = float(nreal)
    m = st[0] / nf
    v = st[1] / nf - m * m
    o = gf_ref[:] * (g2_ref[:] - m) * lax.rsqrt(v + 1e-5) + bf_ref[:]
    out_ref[:] = jnp.where(o >= 0, o, 0.01 * o)


def _edge_scores(bi, bj, exbuf, lane, nh):
    for g in range(CH // 16):
        acc = jnp.zeros((16,), f32)
        for l in range(16):
            r = g * 16 + l
            row = jnp.zeros((16,), f32)
            for hh in range(nh):
                sl = pl.ds(hh * 16, 16)
                d = bi[r, sl] - bj[r, sl]
                row = row + d * d
            row = row + lax.rev(row, (0,))
            sd = row[0]
            for l2 in range(1, 8):
                sd = sd + row[l2]
            acc = jnp.where(lane == l, sd, acc)
        exbuf[pl.ds(g * 16, 16)] = jnp.exp(64.0 - acc)


def _sc_pass1_body(nchp, hdim, s_ref, i_ref, j_ref, ex_ref, den0_ref,
                   den1_ref, iv0, jv0, iv1, jv1, ex0, ex1, bi0, bj0, bi1,
                   bj1, zbuf, is0, is1, gs0, gs1, ss0, ss1, den_sh):
    c_ax = lax.axis_index("c")
    s_ax = lax.axis_index("s")
    wid = s_ax * NC + c_ax
    tslice = den_sh.shape[0] // NS
    off = s_ax * tslice

    def zb_body(k, carry):
        zbuf[pl.ds(k * 16, 16)] = jnp.zeros((16,), f32)
        return carry
    lax.fori_loop(0, tslice // 16, zb_body, 0)
    pltpu.sync_copy(zbuf, den_sh.at[pl.ds(off, tslice)])
    plsc.subcore_barrier()

    cnt = nchp // NWK
    start = wid * cnt
    lane = lax.iota(i32, 16)
    nh = hdim // 16

    def pair(k, carry):
        ciA = start + 2 * k
        ciB = ciA + 1
        dA = [pltpu.async_copy(i_ref.at[ciA], iv0, is0),
              pltpu.async_copy(j_ref.at[ciA], jv0, is0)]
        dB = [pltpu.async_copy(i_ref.at[ciB], iv1, is1),
              pltpu.async_copy(j_ref.at[ciB], jv1, is1)]
        for d in dA:
            d.wait()
        gA = [pltpu.async_copy(s_ref.at[iv0], bi0, gs0),
              pltpu.async_copy(s_ref.at[jv0], bj0, gs0)]
        for d in dB:
            d.wait()
        gB = [pltpu.async_copy(s_ref.at[iv1], bi1, gs1),
              pltpu.async_copy(s_ref.at[jv1], bj1, gs1)]
        for d in gA:
            d.wait()
        _edge_scores(bi0, bj0, ex0, lane, nh)
        pltpu.sync_copy(ex0, ex_ref.at[ciA])
        pltpu.sync_copy(ex0, den_sh.at[iv0], add=True)
        for d in gB:
            d.wait()
        _edge_scores(bi1, bj1, ex1, lane, nh)
        pltpu.sync_copy(ex1, ex_ref.at[ciB])
        pltpu.sync_copy(ex1, den_sh.at[iv1], add=True)
        return carry
    lax.fori_loop(0, cnt // 2, pair, 0)

    plsc.subcore_barrier()
    pltpu.sync_copy(den_sh.at[pl.ds(off, tslice)], zbuf)

    @pl.when(c_ax == 0)
    def _():
        pltpu.sync_copy(zbuf, den0_ref.at[pl.ds(off, tslice)])

    @pl.when(c_ax == 1)
    def _():
        pltpu.sync_copy(zbuf, den1_ref.at[pl.ds(off, tslice)])


def _scale_rows(rows, exv, nh):
    for g in range(CH // 16):
        evec = exv[pl.ds(g * 16, 16)]
        for l in range(16):
            se = evec[l]
            r = g * 16 + l
            for hh in range(nh):
                sl = pl.ds(hh * 16, 16)
                rows[r, sl] = rows[r, sl] * se


def _sc_pass2_body(nchp, hdim, h_ref, ex_ref, i_ref, j_ref, up0_ref, up1_ref,
                   iv0, jv0, ev0, iv1, jv1, ev1, rows0, rows1, zrows,
                   is0, is1, gs0, gs1, ss0, ss1, U_sh):
    c_ax = lax.axis_index("c")
    s_ax = lax.axis_index("s")
    wid = s_ax * NC + c_ax
    tslice = U_sh.shape[0] // NS
    off = s_ax * tslice
    nh = hdim // 16

    for k in range(zrows.shape[0]):
        for hh in range(nh):
            zrows[k, pl.ds(hh * 16, 16)] = jnp.zeros((16,), f32)
    def zc_body(k, carry):
        pltpu.sync_copy(zrows, U_sh.at[pl.ds(off + k * 64, 64)])
        return carry
    lax.fori_loop(0, tslice // 64, zc_body, 0)
    plsc.subcore_barrier()

    cnt = nchp // NWK
    start = wid * cnt

    def pair(k, carry):
        ciA = start + 2 * k
        ciB = ciA + 1
        dA = [pltpu.async_copy(i_ref.at[ciA], iv0, is0),
              pltpu.async_copy(j_ref.at[ciA], jv0, is0),
              pltpu.async_copy(ex_ref.at[ciA], ev0, is0)]
        dB = [pltpu.async_copy(i_ref.at[ciB], iv1, is1),
              pltpu.async_copy(j_ref.at[ciB], jv1, is1),
              pltpu.async_copy(ex_ref.at[ciB], ev1, is1)]
        for d in dA:
            d.wait()
        gA = pltpu.async_copy(h_ref.at[jv0], rows0, gs0)
        for d in dB:
            d.wait()
        gB = pltpu.async_copy(h_ref.at[jv1], rows1, gs1)
        gA.wait()
        _scale_rows(rows0, ev0, nh)
        pltpu.sync_copy(rows0, U_sh.at[iv0], add=True)
        gB.wait()
        _scale_rows(rows1, ev1, nh)
        pltpu.sync_copy(rows1, U_sh.at[iv1], add=True)
        return carry
    lax.fori_loop(0, cnt // 2, pair, 0)

    plsc.subcore_barrier()

    def dr_body(k, carry):
        pltpu.sync_copy(U_sh.at[pl.ds(off + k * 64, 64)], zrows)

        @pl.when(c_ax == 0)
        def _():
            pltpu.sync_copy(zrows, up0_ref.at[pl.ds(off + k * 64, 64)])

        @pl.when(c_ax == 1)
        def _():
            pltpu.sync_copy(zrows, up1_ref.at[pl.ds(off + k * 64, 64)])
        return carry
    lax.fori_loop(0, tslice // 64, dr_body, 0)


def kernel(x, y, pos, edge_index, W_unary, g_unary, b_unary, W_pair, g_pair,
           b_pair, c, W_mlp, g_mlp, b_mlp, W_fuse, g_fuse, b_fuse):
    n, d = x.shape
    e = edge_index.shape[1]
    h = W_unary.shape[1]
    out_d = W_mlp.shape[1]
    assert e % CH == 0
    np2 = _cdiv(n, 1024) * 1024
    rrows = 1024
    nb = np2 // rrows
    nch = e // CH
    SDS = jax.ShapeDtypeStruct

    xp = jnp.pad(x, ((0, np2 - n), (0, 0)))
    yp = jnp.pad(y, ((0, np2 - n), (0, 0)))
    nchp = _cdiv(nch, 2 * NWK) * (2 * NWK)
    epad = nchp * CH - e
    i2 = jnp.concatenate(
        [edge_index[0], jnp.full((epad,), np2 - 1, i32)]).reshape(nchp, CH)
    j2 = jnp.concatenate(
        [edge_index[1], jnp.zeros((epad,), i32)]).reshape(nchp, CH)
    wf_h = W_fuse[:out_d]
    wf_y = W_fuse[out_d:]

    bc, cb = pl.pallas_call(
        _prep_w_body,
        out_shape=[SDS((h, h), f32), SDS((h, h), f32)],
    )(c)

    rowblk = lambda bdim: pl.BlockSpec((rrows, bdim), lambda b: (b, 0))
    full2 = lambda s0, s1: pl.BlockSpec((s0, s1), lambda b: (0, 0))
    full1 = lambda s0: pl.BlockSpec((s0,), lambda b: (0,))
    statblk = lambda k: pl.BlockSpec((1, 1, k, h), lambda b: (b, 0, 0, 0))
    statall = lambda k: pl.BlockSpec((nb, 1, k, h), lambda b: (0, 0, 0, 0))
    statblk_o = lambda k: pl.BlockSpec((1, 1, k, out_d), lambda b: (b, 0, 0, 0))
    statall_o = lambda k: pl.BlockSpec((nb, 1, k, out_d), lambda b: (0, 0, 0, 0))

    p1, p2, s1 = pl.pallas_call(
        _pre_body,
        grid=(nb,),
        in_specs=[rowblk(d), rowblk(d), full2(d, h), full2(d, h)],
        out_specs=[rowblk(h), rowblk(h), statblk(4)],
        out_shape=[SDS((np2, h), f32), SDS((np2, h), f32),
                   SDS((nb, 1, 4, h), f32)],
    )(xp, yp, W_unary, W_pair)

    s_nodes, zb, h0 = pl.pallas_call(
        functools.partial(_norm_body, n),
        grid=(nb,),
        in_specs=[rowblk(h), rowblk(h), statall(4), full1(h), full1(h),
                  full1(h), full1(h), full2(h, h)],
        out_specs=[rowblk(h), rowblk(h), rowblk(h)],
        out_shape=[SDS((np2, h), f32), SDS((np2, h), f32), SDS((np2, h), f32)],
    )(p1, p2, s1, g_unary, b_unary, g_pair, b_pair, bc)

    mesh = plsc.VectorSubcoreMesh(core_axis_name="c", subcore_axis_name="s",
                                  num_cores=NC, num_subcores=NS)
    sems6 = [pltpu.SemaphoreType.DMA] * 6
    ex2, den0, den1 = pl.kernel(
        functools.partial(_sc_pass1_body, nchp, h),
        out_type=[SDS((nchp, CH), f32), SDS((np2,), f32), SDS((np2,), f32)],
        mesh=mesh,
        compiler_params=pltpu.CompilerParams(use_tc_tiling_on_sc=False),
        scratch_types=[
            pltpu.VMEM((CH,), i32),
            pltpu.VMEM((CH,), i32),
            pltpu.VMEM((CH,), i32),
            pltpu.VMEM((CH,), i32),
            pltpu.VMEM((CH,), f32),
            pltpu.VMEM((CH,), f32),
            pltpu.VMEM((CH, h), f32),
            pltpu.VMEM((CH, h), f32),
            pltpu.VMEM((CH, h), f32),
            pltpu.VMEM((CH, h), f32),
            pltpu.VMEM((np2 // NS,), f32),
        ] + sems6 + [
            pltpu.VMEM_SHARED((np2,), f32),
        ],
    )(s_nodes, i2, j2)
    den0 = den0.reshape(np2, 1)
    den1 = den1.reshape(np2, 1)

    hcur = h0
    for _ in range(2):
        up0, up1 = pl.kernel(
            functools.partial(_sc_pass2_body, nchp, h),
            out_type=[SDS((np2, h), f32), SDS((np2, h), f32)],
            mesh=mesh,
            compiler_params=pltpu.CompilerParams(use_tc_tiling_on_sc=False),
            scratch_types=[
                pltpu.VMEM((CH,), i32),
                pltpu.VMEM((CH,), i32),
                pltpu.VMEM((CH,), f32),
                pltpu.VMEM((CH,), i32),
                pltpu.VMEM((CH,), i32),
                pltpu.VMEM((CH,), f32),
                pltpu.VMEM((CH, h), f32),
                pltpu.VMEM((CH, h), f32),
                pltpu.VMEM((64, h), f32),
            ] + sems6 + [
                pltpu.VMEM_SHARED((np2, h), f32),
            ],
        )(hcur, ex2, i2, j2)

        colblk = pl.BlockSpec((rrows, 1), lambda b: (b, 0))
        hcur = pl.pallas_call(
            functools.partial(_update_body, n, rrows),
            grid=(nb,),
            in_specs=[rowblk(h), rowblk(h), colblk, colblk,
                      rowblk(h), full2(h, h)],
            out_specs=rowblk(h),
            out_shape=SDS((np2, h), f32),
        )(up0, up1, den0, den1, zb, cb)

    g1, s2 = pl.pallas_call(
        _posta_body,
        grid=(nb,),
        in_specs=[rowblk(h), full2(h, out_d)],
        out_specs=[rowblk(out_d), statblk_o(2)],
        out_shape=[SDS((np2, out_d), f32), SDS((nb, 1, 2, out_d), f32)],
    )(hcur, W_mlp)

    g2, s3 = pl.pallas_call(
        functools.partial(_postb_body, n, rrows),
        grid=(nb,),
        in_specs=[rowblk(out_d), statall_o(2), full1(out_d), full1(out_d),
                  rowblk(d), full2(out_d, out_d), full2(d, out_d)],
        out_specs=[rowblk(out_d), statblk_o(2)],
        out_shape=[SDS((np2, out_d), f32), SDS((nb, 1, 2, out_d), f32)],
    )(g1, s2, g_mlp, b_mlp, yp, wf_h, wf_y)

    outp = pl.pallas_call(
        functools.partial(_postc_body, n),
        grid=(nb,),
        in_specs=[rowblk(out_d), statall_o(2), full1(out_d), full1(out_d)],
        out_specs=rowblk(out_d),
        out_shape=SDS((np2, out_d), f32),
    )(g2, s3, g_fuse, b_fuse)

    return outp[:n]


# speedup vs baseline: 13.9049x; 1.0662x over previous
import functools

import jax
import jax.numpy as jnp
from jax import lax
from jax.experimental import pallas as pl
from jax.experimental.pallas import tpu as pltpu
from jax.experimental.pallas import tpu_sc as plsc

NC = 2
NS = 16
NWK = NC * NS
CH = 128

f32 = jnp.float32
i32 = jnp.int32


def _cdiv(a, b):
    return (a + b - 1) // b


def _prep_w_body(c_ref, bc_ref, cb_ref):
    cc = c_ref[:]
    C = lax.dot_general(cc, cc, (((0,), (0,)), ((), ())),
                        preferred_element_type=f32)
    h = C.shape[0]
    ii = lax.broadcasted_iota(i32, (h, h), 0)
    jj = lax.broadcasted_iota(i32, (h, h), 1)
    I = jnp.where(ii == jj, 1.0, 0.0).astype(f32)
    A = I + C
    X = 0.5 * I
    for _ in range(12):
        AX = jnp.dot(A, X, preferred_element_type=f32)
        X = jnp.dot(X, 2.0 * I - AX, preferred_element_type=f32)
    bc_ref[:] = X
    cb_ref[:] = jnp.dot(C, X, preferred_element_type=f32)


def _pre_body(x_ref, y_ref, wu_ref, wp_ref, p1_ref, p2_ref, s_ref):
    p1 = jnp.dot(x_ref[:], wu_ref[:], preferred_element_type=f32)
    p2 = jnp.dot(y_ref[:], wp_ref[:], preferred_element_type=f32)
    p1_ref[:] = p1
    p2_ref[:] = p2
    s_ref[0, 0, 0, :] = jnp.sum(p1, axis=0)
    s_ref[0, 0, 1, :] = jnp.sum(p1 * p1, axis=0)
    s_ref[0, 0, 2, :] = jnp.sum(p2, axis=0)
    s_ref[0, 0, 3, :] = jnp.sum(p2 * p2, axis=0)


def _norm_body(nreal, p1_ref, p2_ref, s_ref, gu_ref, bu_ref, gp_ref, bp_ref,
               bc_ref, sn_ref, zb_ref, h0_ref):
    st = jnp.sum(s_ref[:, 0, :, :], axis=0)
    nf = float(nreal)
    m1 = st[0] / nf
    v1 = st[1] / nf - m1 * m1
    m2 = st[2] / nf
    v2 = st[3] / nf - m2 * m2
    xh = gu_ref[:] * (p1_ref[:] - m1) * lax.rsqrt(v1 + 1e-5) + bu_ref[:]
    sn = gp_ref[:] * (p2_ref[:] - m2) * lax.rsqrt(v2 + 1e-5) + bp_ref[:]
    sn_ref[:] = sn
    h0_ref[:] = xh
    zb_ref[:] = jnp.dot(xh, bc_ref[:], preferred_element_type=f32)


def _update_body(nreal, rrows, up0_ref, up1_ref, den0_ref, den1_ref, zb_ref,
                 cb_ref, h_ref):
    U = up0_ref[:] + up1_ref[:]
    den = den0_ref[:] + den1_ref[:]
    safe = jnp.where(den > 0, den, 1.0)
    agg = jnp.where(den > 0, U / safe, 0.0)
    h = zb_ref[:] + jnp.dot(agg, cb_ref[:], preferred_element_type=f32)
    rid = pl.program_id(0) * rrows + lax.broadcasted_iota(i32, (rrows, 1), 0)
    h_ref[:] = jnp.where(rid < nreal, h, 0.0)


def _posta_body(h_ref, wm_ref, g1_ref, s_ref):
    g1 = jnp.dot(h_ref[:], wm_ref[:], preferred_element_type=f32)
    g1_ref[:] = g1
    s_ref[0, 0, 0, :] = jnp.sum(g1, axis=0)
    s_ref[0, 0, 1, :] = jnp.sum(g1 * g1, axis=0)


def _postb_body(nreal, rrows, g1_ref, s_ref, gm_ref, bm_ref, y_ref, wfh_ref,
                wfy_ref, g2_ref, s2_ref):
    st = jnp.sum(s_ref[:, 0, :, :], axis=0)
    nf = float(nreal)
    m = st[0] / nf
    v = st[1] / nf - m * m
    hm = gm_ref[:] * (g1_ref[:] - m) * lax.rsqrt(v + 1e-5) + bm_ref[:]
    hm = jnp.where(hm >= 0, hm, 0.01 * hm)
    rid = pl.program_id(0) * rrows + lax.broadcasted_iota(i32, (rrows, 1), 0)
    hm = jnp.where(rid < nreal, hm, 0.0)
    g2 = (jnp.dot(hm, wfh_ref[:], preferred_element_type=f32)
          + jnp.dot(y_ref[:], wfy_ref[:], preferred_element_type=f32))
    g2_ref[:] = g2
    s2_ref[0, 0, 0, :] = jnp.sum(g2, axis=0)
    s2_ref[0, 0, 1, :] = jnp.sum(g2 * g2, axis=0)


def _postc_body(nreal, g2_ref, s_ref, gf_ref, bf_ref, out_ref):
    st = jnp.sum(s_ref[:, 0, :, :], axis=0)
    nf = float(nreal)
    m = st[0] / nf
    v = st[1] / nf - m * m
    o = gf_ref[:] * (g2_ref[:] - m) * lax.rsqrt(v + 1e-5) + bf_ref[:]
    out_ref[:] = jnp.where(o >= 0, o, 0.01 * o)


def _edge_scores(bi, bj, exbuf, lane, nh):
    for g in range(CH // 16):
        masked = []
        for l in range(16):
            r = g * 16 + l
            parts = []
            for hh in range(nh):
                sl = pl.ds(hh * 16, 16)
                d = bi[r, sl] - bj[r, sl]
                parts.append(d * d)
            row = parts[0]
            for p in parts[1:]:
                row = row + p
            row = row + lax.rev(row, (0,))
            sd = (((row[0] + row[1]) + (row[2] + row[3]))
                  + ((row[4] + row[5]) + (row[6] + row[7])))
            masked.append(jnp.where(lane == l, sd, 0.0))
        while len(masked) > 1:
            masked = [a + b for a, b in zip(masked[::2], masked[1::2])]
        exbuf[pl.ds(g * 16, 16)] = jnp.exp(64.0 - masked[0])


def _sc_pass1_body(nchp, hdim, s_ref, i_ref, j_ref, ex_ref, den0_ref,
                   den1_ref, iv0, jv0, iv1, jv1, ex0, ex1, bi0, bj0, bi1,
                   bj1, zbuf, is0, is1, gs0, gs1, ss0, ss1, den_sh):
    c_ax = lax.axis_index("c")
    s_ax = lax.axis_index("s")
    wid = s_ax * NC + c_ax
    tslice = den_sh.shape[0] // NS
    off = s_ax * tslice

    def zb_body(k, carry):
        zbuf[pl.ds(k * 16, 16)] = jnp.zeros((16,), f32)
        return carry
    lax.fori_loop(0, tslice // 16, zb_body, 0)
    pltpu.sync_copy(zbuf, den_sh.at[pl.ds(off, tslice)])
    plsc.subcore_barrier()

    cnt = nchp // NWK
    start = wid * cnt
    lane = lax.iota(i32, 16)
    nh = hdim // 16

    def pair(k, carry):
        ciA = start + 2 * k
        ciB = ciA + 1
        dA = [pltpu.async_copy(i_ref.at[ciA], iv0, is0),
              pltpu.async_copy(j_ref.at[ciA], jv0, is0)]
        dB = [pltpu.async_copy(i_ref.at[ciB], iv1, is1),
              pltpu.async_copy(j_ref.at[ciB], jv1, is1)]
        for d in dA:
            d.wait()
        gA = [pltpu.async_copy(s_ref.at[iv0], bi0, gs0),
              pltpu.async_copy(s_ref.at[jv0], bj0, gs0)]
        for d in dB:
            d.wait()
        gB = [pltpu.async_copy(s_ref.at[iv1], bi1, gs1),
              pltpu.async_copy(s_ref.at[jv1], bj1, gs1)]
        for d in gA:
            d.wait()
        _edge_scores(bi0, bj0, ex0, lane, nh)
        pltpu.sync_copy(ex0, ex_ref.at[ciA])
        pltpu.sync_copy(ex0, den_sh.at[iv0], add=True)
        for d in gB:
            d.wait()
        _edge_scores(bi1, bj1, ex1, lane, nh)
        pltpu.sync_copy(ex1, ex_ref.at[ciB])
        pltpu.sync_copy(ex1, den_sh.at[iv1], add=True)
        return carry
    lax.fori_loop(0, cnt // 2, pair, 0)

    plsc.subcore_barrier()
    pltpu.sync_copy(den_sh.at[pl.ds(off, tslice)], zbuf)

    @pl.when(c_ax == 0)
    def _():
        pltpu.sync_copy(zbuf, den0_ref.at[pl.ds(off, tslice)])

    @pl.when(c_ax == 1)
    def _():
        pltpu.sync_copy(zbuf, den1_ref.at[pl.ds(off, tslice)])


def _scale_rows(rows, exv, nh):
    for g in range(CH // 16):
        evec = exv[pl.ds(g * 16, 16)]
        for l in range(16):
            se = evec[l]
            r = g * 16 + l
            for hh in range(nh):
                sl = pl.ds(hh * 16, 16)
                rows[r, sl] = rows[r, sl] * se


def _sc_pass2_body(nchp, hdim, h_ref, ex_ref, i_ref, j_ref, up0_ref, up1_ref,
                   iv0, jv0, ev0, iv1, jv1, ev1, rows0, rows1, zrows,
                   is0, is1, gs0, gs1, ss0, ss1, U_sh):
    c_ax = lax.axis_index("c")
    s_ax = lax.axis_index("s")
    wid = s_ax * NC + c_ax
    tslice = U_sh.shape[0] // NS
    off = s_ax * tslice
    nh = hdim // 16

    for k in range(zrows.shape[0]):
        for hh in range(nh):
            zrows[k, pl.ds(hh * 16, 16)] = jnp.zeros((16,), f32)
    def zc_body(k, carry):
        pltpu.sync_copy(zrows, U_sh.at[pl.ds(off + k * 64, 64)])
        return carry
    lax.fori_loop(0, tslice // 64, zc_body, 0)
    plsc.subcore_barrier()

    cnt = nchp // NWK
    start = wid * cnt

    def pair(k, carry):
        ciA = start + 2 * k
        ciB = ciA + 1
        dA = [pltpu.async_copy(i_ref.at[ciA], iv0, is0),
              pltpu.async_copy(j_ref.at[ciA], jv0, is0),
              pltpu.async_copy(ex_ref.at[ciA], ev0, is0)]
        dB = [pltpu.async_copy(i_ref.at[ciB], iv1, is1),
              pltpu.async_copy(j_ref.at[ciB], jv1, is1),
              pltpu.async_copy(ex_ref.at[ciB], ev1, is1)]
        for d in dA:
            d.wait()
        gA = pltpu.async_copy(h_ref.at[jv0], rows0, gs0)
        for d in dB:
            d.wait()
        gB = pltpu.async_copy(h_ref.at[jv1], rows1, gs1)
        gA.wait()
        _scale_rows(rows0, ev0, nh)
        pltpu.sync_copy(rows0, U_sh.at[iv0], add=True)
        gB.wait()
        _scale_rows(rows1, ev1, nh)
        pltpu.sync_copy(rows1, U_sh.at[iv1], add=True)
        return carry
    lax.fori_loop(0, cnt // 2, pair, 0)

    plsc.subcore_barrier()

    def dr_body(k, carry):
        pltpu.sync_copy(U_sh.at[pl.ds(off + k * 64, 64)], zrows)

        @pl.when(c_ax == 0)
        def _():
            pltpu.sync_copy(zrows, up0_ref.at[pl.ds(off + k * 64, 64)])

        @pl.when(c_ax == 1)
        def _():
            pltpu.sync_copy(zrows, up1_ref.at[pl.ds(off + k * 64, 64)])
        return carry
    lax.fori_loop(0, tslice // 64, dr_body, 0)


def kernel(x, y, pos, edge_index, W_unary, g_unary, b_unary, W_pair, g_pair,
           b_pair, c, W_mlp, g_mlp, b_mlp, W_fuse, g_fuse, b_fuse):
    n, d = x.shape
    e = edge_index.shape[1]
    h = W_unary.shape[1]
    out_d = W_mlp.shape[1]
    assert e % CH == 0
    np2 = _cdiv(n, 1024) * 1024
    rrows = 1024
    nb = np2 // rrows
    nch = e // CH
    SDS = jax.ShapeDtypeStruct

    xp = jnp.pad(x, ((0, np2 - n), (0, 0)))
    yp = jnp.pad(y, ((0, np2 - n), (0, 0)))
    nchp = _cdiv(nch, 2 * NWK) * (2 * NWK)
    epad = nchp * CH - e
    i2 = jnp.concatenate(
        [edge_index[0], jnp.full((epad,), np2 - 1, i32)]).reshape(nchp, CH)
    j2 = jnp.concatenate(
        [edge_index[1], jnp.zeros((epad,), i32)]).reshape(nchp, CH)
    wf_h = W_fuse[:out_d]
    wf_y = W_fuse[out_d:]

    bc, cb = pl.pallas_call(
        _prep_w_body,
        out_shape=[SDS((h, h), f32), SDS((h, h), f32)],
    )(c)

    rowblk = lambda bdim: pl.BlockSpec((rrows, bdim), lambda b: (b, 0))
    full2 = lambda s0, s1: pl.BlockSpec((s0, s1), lambda b: (0, 0))
    full1 = lambda s0: pl.BlockSpec((s0,), lambda b: (0,))
    statblk = lambda k: pl.BlockSpec((1, 1, k, h), lambda b: (b, 0, 0, 0))
    statall = lambda k: pl.BlockSpec((nb, 1, k, h), lambda b: (0, 0, 0, 0))
    statblk_o = lambda k: pl.BlockSpec((1, 1, k, out_d), lambda b: (b, 0, 0, 0))
    statall_o = lambda k: pl.BlockSpec((nb, 1, k, out_d), lambda b: (0, 0, 0, 0))

    p1, p2, s1 = pl.pallas_call(
        _pre_body,
        grid=(nb,),
        in_specs=[rowblk(d), rowblk(d), full2(d, h), full2(d, h)],
        out_specs=[rowblk(h), rowblk(h), statblk(4)],
        out_shape=[SDS((np2, h), f32), SDS((np2, h), f32),
                   SDS((nb, 1, 4, h), f32)],
    )(xp, yp, W_unary, W_pair)

    s_nodes, zb, h0 = pl.pallas_call(
        functools.partial(_norm_body, n),
        grid=(nb,),
        in_specs=[rowblk(h), rowblk(h), statall(4), full1(h), full1(h),
                  full1(h), full1(h), full2(h, h)],
        out_specs=[rowblk(h), rowblk(h), rowblk(h)],
        out_shape=[SDS((np2, h), f32), SDS((np2, h), f32), SDS((np2, h), f32)],
    )(p1, p2, s1, g_unary, b_unary, g_pair, b_pair, bc)

    mesh = plsc.VectorSubcoreMesh(core_axis_name="c", subcore_axis_name="s",
                                  num_cores=NC, num_subcores=NS)
    sems6 = [pltpu.SemaphoreType.DMA] * 6
    ex2, den0, den1 = pl.kernel(
        functools.partial(_sc_pass1_body, nchp, h),
        out_type=[SDS((nchp, CH), f32), SDS((np2,), f32), SDS((np2,), f32)],
        mesh=mesh,
        compiler_params=pltpu.CompilerParams(use_tc_tiling_on_sc=False),
        scratch_types=[
            pltpu.VMEM((CH,), i32),
            pltpu.VMEM((CH,), i32),
            pltpu.VMEM((CH,), i32),
            pltpu.VMEM((CH,), i32),
            pltpu.VMEM((CH,), f32),
            pltpu.VMEM((CH,), f32),
            pltpu.VMEM((CH, h), f32),
            pltpu.VMEM((CH, h), f32),
            pltpu.VMEM((CH, h), f32),
            pltpu.VMEM((CH, h), f32),
            pltpu.VMEM((np2 // NS,), f32),
        ] + sems6 + [
            pltpu.VMEM_SHARED((np2,), f32),
        ],
    )(s_nodes, i2, j2)
    den0 = den0.reshape(np2, 1)
    den1 = den1.reshape(np2, 1)

    hcur = h0
    for _ in range(2):
        up0, up1 = pl.kernel(
            functools.partial(_sc_pass2_body, nchp, h),
            out_type=[SDS((np2, h), f32), SDS((np2, h), f32)],
            mesh=mesh,
            compiler_params=pltpu.CompilerParams(use_tc_tiling_on_sc=False),
            scratch_types=[
                pltpu.VMEM((CH,), i32),
                pltpu.VMEM((CH,), i32),
                pltpu.VMEM((CH,), f32),
                pltpu.VMEM((CH,), i32),
                pltpu.VMEM((CH,), i32),
                pltpu.VMEM((CH,), f32),
                pltpu.VMEM((CH, h), f32),
                pltpu.VMEM((CH, h), f32),
                pltpu.VMEM((64, h), f32),
            ] + sems6 + [
                pltpu.VMEM_SHARED((np2, h), f32),
            ],
        )(hcur, ex2, i2, j2)

        colblk = pl.BlockSpec((rrows, 1), lambda b: (b, 0))
        hcur = pl.pallas_call(
            functools.partial(_update_body, n, rrows),
            grid=(nb,),
            in_specs=[rowblk(h), rowblk(h), colblk, colblk,
                      rowblk(h), full2(h, h)],
            out_specs=rowblk(h),
            out_shape=SDS((np2, h), f32),
        )(up0, up1, den0, den1, zb, cb)

    g1, s2 = pl.pallas_call(
        _posta_body,
        grid=(nb,),
        in_specs=[rowblk(h), full2(h, out_d)],
        out_specs=[rowblk(out_d), statblk_o(2)],
        out_shape=[SDS((np2, out_d), f32), SDS((nb, 1, 2, out_d), f32)],
    )(hcur, W_mlp)

    g2, s3 = pl.pallas_call(
        functools.partial(_postb_body, n, rrows),
        grid=(nb,),
        in_specs=[rowblk(out_d), statall_o(2), full1(out_d), full1(out_d),
                  rowblk(d), full2(out_d, out_d), full2(d, out_d)],
        out_specs=[rowblk(out_d), statblk_o(2)],
        out_shape=[SDS((np2, out_d), f32), SDS((nb, 1, 2, out_d), f32)],
    )(g1, s2, g_mlp, b_mlp, yp, wf_h, wf_y)

    outp = pl.pallas_call(
        functools.partial(_postc_body, n),
        grid=(nb,),
        in_specs=[rowblk(out_d), statall_o(2), full1(out_d), full1(out_d)],
        out_specs=rowblk(out_d),
        out_shape=SDS((np2, out_d), f32),
    )(g2, s3, g_fuse, b_fuse)

    return outp[:n]


# speedup vs baseline: 14.2203x; 1.0227x over previous
import functools

import jax
import jax.numpy as jnp
from jax import lax
from jax.experimental import pallas as pl
from jax.experimental.pallas import tpu as pltpu
from jax.experimental.pallas import tpu_sc as plsc

NC = 2
NS = 16
NWK = NC * NS
CH = 128

f32 = jnp.float32
i32 = jnp.int32


def _cdiv(a, b):
    return (a + b - 1) // b


def _prep_w_body(c_ref, bc_ref, cb_ref):
    cc = c_ref[:]
    C = lax.dot_general(cc, cc, (((0,), (0,)), ((), ())),
                        preferred_element_type=f32)
    h = C.shape[0]
    ii = lax.broadcasted_iota(i32, (h, h), 0)
    jj = lax.broadcasted_iota(i32, (h, h), 1)
    I = jnp.where(ii == jj, 1.0, 0.0).astype(f32)
    A = I + C
    X = 0.5 * I
    for _ in range(12):
        AX = jnp.dot(A, X, preferred_element_type=f32)
        X = jnp.dot(X, 2.0 * I - AX, preferred_element_type=f32)
    bc_ref[:] = X
    cb_ref[:] = jnp.dot(C, X, preferred_element_type=f32)


def _pre_body(x_ref, y_ref, wu_ref, wp_ref, p1_ref, p2_ref, s_ref):
    p1 = jnp.dot(x_ref[:], wu_ref[:], preferred_element_type=f32)
    p2 = jnp.dot(y_ref[:], wp_ref[:], preferred_element_type=f32)
    p1_ref[:] = p1
    p2_ref[:] = p2
    s_ref[0, 0, 0, :] = jnp.sum(p1, axis=0)
    s_ref[0, 0, 1, :] = jnp.sum(p1 * p1, axis=0)
    s_ref[0, 0, 2, :] = jnp.sum(p2, axis=0)
    s_ref[0, 0, 3, :] = jnp.sum(p2 * p2, axis=0)


def _norm_body(nreal, p1_ref, p2_ref, s_ref, gu_ref, bu_ref, gp_ref, bp_ref,
               bc_ref, sn_ref, zb_ref, h0_ref):
    st = jnp.sum(s_ref[:, 0, :, :], axis=0)
    nf = float(nreal)
    m1 = st[0] / nf
    v1 = st[1] / nf - m1 * m1
    m2 = st[2] / nf
    v2 = st[3] / nf - m2 * m2
    xh = gu_ref[:] * (p1_ref[:] - m1) * lax.rsqrt(v1 + 1e-5) + bu_ref[:]
    sn = gp_ref[:] * (p2_ref[:] - m2) * lax.rsqrt(v2 + 1e-5) + bp_ref[:]
    sn_ref[:] = sn
    h0_ref[:] = xh
    zb_ref[:] = jnp.dot(xh, bc_ref[:], preferred_element_type=f32)


def _update_body(nreal, rrows, up0_ref, up1_ref, den0_ref, den1_ref, zb_ref,
                 cb_ref, h_ref):
    U = up0_ref[:] + up1_ref[:]
    den = den0_ref[:] + den1_ref[:]
    safe = jnp.where(den > 0, den, 1.0)
    agg = jnp.where(den > 0, U / safe, 0.0)
    h = zb_ref[:] + jnp.dot(agg, cb_ref[:], preferred_element_type=f32)
    rid = pl.program_id(0) * rrows + lax.broadcasted_iota(i32, (rrows, 1), 0)
    h_ref[:] = jnp.where(rid < nreal, h, 0.0)


def _posta_body(h_ref, wm_ref, g1_ref, s_ref):
    g1 = jnp.dot(h_ref[:], wm_ref[:], preferred_element_type=f32)
    g1_ref[:] = g1
    s_ref[0, 0, 0, :] = jnp.sum(g1, axis=0)
    s_ref[0, 0, 1, :] = jnp.sum(g1 * g1, axis=0)


def _postb_body(nreal, rrows, g1_ref, s_ref, gm_ref, bm_ref, y_ref, wfh_ref,
                wfy_ref, g2_ref, s2_ref):
    st = jnp.sum(s_ref[:, 0, :, :], axis=0)
    nf = float(nreal)
    m = st[0] / nf
    v = st[1] / nf - m * m
    hm = gm_ref[:] * (g1_ref[:] - m) * lax.rsqrt(v + 1e-5) + bm_ref[:]
    hm = jnp.where(hm >= 0, hm, 0.01 * hm)
    rid = pl.program_id(0) * rrows + lax.broadcasted_iota(i32, (rrows, 1), 0)
    hm = jnp.where(rid < nreal, hm, 0.0)
    g2 = (jnp.dot(hm, wfh_ref[:], preferred_element_type=f32)
          + jnp.dot(y_ref[:], wfy_ref[:], preferred_element_type=f32))
    g2_ref[:] = g2
    s2_ref[0, 0, 0, :] = jnp.sum(g2, axis=0)
    s2_ref[0, 0, 1, :] = jnp.sum(g2 * g2, axis=0)


def _postc_body(nreal, g2_ref, s_ref, gf_ref, bf_ref, out_ref):
    st = jnp.sum(s_ref[:, 0, :, :], axis=0)
    nf = float(nreal)
    m = st[0] / nf
    v = st[1] / nf - m * m
    o = gf_ref[:] * (g2_ref[:] - m) * lax.rsqrt(v + 1e-5) + bf_ref[:]
    out_ref[:] = jnp.where(o >= 0, o, 0.01 * o)


def _edge_scores(bi, bj, exbuf, lane, nh):
    for g in range(CH // 16):
        masked = []
        for l in range(16):
            r = g * 16 + l
            parts = []
            for hh in range(nh):
                sl = pl.ds(hh * 16, 16)
                d = bi[r, sl] - bj[r, sl]
                parts.append(d * d)
            row = parts[0]
            for p in parts[1:]:
                row = row + p
            row = row + lax.rev(row, (0,))
            sd = (((row[0] + row[1]) + (row[2] + row[3]))
                  + ((row[4] + row[5]) + (row[6] + row[7])))
            masked.append(jnp.where(lane == l, sd, 0.0))
        while len(masked) > 1:
            masked = [a + b for a, b in zip(masked[::2], masked[1::2])]
        exbuf[pl.ds(g * 16, 16)] = jnp.exp(64.0 - masked[0])


def _sc_pass1_body(nchp, hdim, s_ref, i_ref, j_ref, ex_ref, den0_ref,
                   den1_ref, iv0, jv0, iv1, jv1, ex0, ex1, bi0, bj0, bi1,
                   bj1, zbuf, is0, is1, gs0, gs1, ss0, ss1, den_sh):
    c_ax = lax.axis_index("c")
    s_ax = lax.axis_index("s")
    wid = s_ax * NC + c_ax
    tslice = den_sh.shape[0] // NS
    off = s_ax * tslice

    def zb_body(k, carry):
        zbuf[pl.ds(k * 16, 16)] = jnp.zeros((16,), f32)
        return carry
    lax.fori_loop(0, tslice // 16, zb_body, 0)
    pltpu.sync_copy(zbuf, den_sh.at[pl.ds(off, tslice)])
    plsc.subcore_barrier()

    cnt = nchp // NWK
    start = wid * cnt
    lane = lax.iota(i32, 16)
    nh = hdim // 16

    def pair(k, carry):
        ciA = start + 2 * k
        ciB = ciA + 1
        dA = [pltpu.async_copy(i_ref.at[ciA], iv0, is0),
              pltpu.async_copy(j_ref.at[ciA], jv0, is0)]
        dB = [pltpu.async_copy(i_ref.at[ciB], iv1, is1),
              pltpu.async_copy(j_ref.at[ciB], jv1, is1)]
        for d in dA:
            d.wait()
        gA = [pltpu.async_copy(s_ref.at[iv0], bi0, gs0),
              pltpu.async_copy(s_ref.at[jv0], bj0, gs0)]
        for d in dB:
            d.wait()
        gB = [pltpu.async_copy(s_ref.at[iv1], bi1, gs1),
              pltpu.async_copy(s_ref.at[jv1], bj1, gs1)]
        for d in gA:
            d.wait()
        _edge_scores(bi0, bj0, ex0, lane, nh)
        wA = pltpu.async_copy(ex0, ex_ref.at[ciA], ss0)
        pltpu.sync_copy(ex0, den_sh.at[iv0], add=True)
        for d in gB:
            d.wait()
        _edge_scores(bi1, bj1, ex1, lane, nh)
        wB = pltpu.async_copy(ex1, ex_ref.at[ciB], ss1)
        pltpu.sync_copy(ex1, den_sh.at[iv1], add=True)
        wA.wait()
        wB.wait()
        return carry
    lax.fori_loop(0, cnt // 2, pair, 0)

    plsc.subcore_barrier()
    pltpu.sync_copy(den_sh.at[pl.ds(off, tslice)], zbuf)

    @pl.when(c_ax == 0)
    def _():
        pltpu.sync_copy(zbuf, den0_ref.at[pl.ds(off, tslice)])

    @pl.when(c_ax == 1)
    def _():
        pltpu.sync_copy(zbuf, den1_ref.at[pl.ds(off, tslice)])


def _scale_rows(rows, exv, nh):
    for g in range(CH // 16):
        evec = exv[pl.ds(g * 16, 16)]
        for l in range(16):
            se = evec[l]
            r = g * 16 + l
            for hh in range(nh):
                sl = pl.ds(hh * 16, 16)
                rows[r, sl] = rows[r, sl] * se


def _sc_pass2_body(nchp, hdim, h_ref, ex_ref, i_ref, j_ref, up0_ref, up1_ref,
                   iv0, jv0, ev0, iv1, jv1, ev1, rows0, rows1, zrows,
                   is0, is1, gs0, gs1, ss0, ss1, U_sh):
    c_ax = lax.axis_index("c")
    s_ax = lax.axis_index("s")
    wid = s_ax * NC + c_ax
    tslice = U_sh.shape[0] // NS
    off = s_ax * tslice
    nh = hdim // 16

    for k in range(zrows.shape[0]):
        for hh in range(nh):
            zrows[k, pl.ds(hh * 16, 16)] = jnp.zeros((16,), f32)
    def zc_body(k, carry):
        pltpu.sync_copy(zrows, U_sh.at[pl.ds(off + k * 64, 64)])
        return carry
    lax.fori_loop(0, tslice // 64, zc_body, 0)
    plsc.subcore_barrier()

    cnt = nchp // NWK
    start = wid * cnt

    def pair(k, carry):
        ciA = start + 2 * k
        ciB = ciA + 1
        dA = [pltpu.async_copy(i_ref.at[ciA], iv0, is0),
              pltpu.async_copy(j_ref.at[ciA], jv0, is0),
              pltpu.async_copy(ex_ref.at[ciA], ev0, is0)]
        dB = [pltpu.async_copy(i_ref.at[ciB], iv1, is1),
              pltpu.async_copy(j_ref.at[ciB], jv1, is1),
              pltpu.async_copy(ex_ref.at[ciB], ev1, is1)]
        for d in dA:
            d.wait()
        gA = pltpu.async_copy(h_ref.at[jv0], rows0, gs0)
        for d in dB:
            d.wait()
        gB = pltpu.async_copy(h_ref.at[jv1], rows1, gs1)
        gA.wait()
        _scale_rows(rows0, ev0, nh)
        pltpu.sync_copy(rows0, U_sh.at[iv0], add=True)
        gB.wait()
        _scale_rows(rows1, ev1, nh)
        pltpu.sync_copy(rows1, U_sh.at[iv1], add=True)
        return carry
    lax.fori_loop(0, cnt // 2, pair, 0)

    plsc.subcore_barrier()

    def dr_body(k, carry):
        pltpu.sync_copy(U_sh.at[pl.ds(off + k * 64, 64)], zrows)

        @pl.when(c_ax == 0)
        def _():
            pltpu.sync_copy(zrows, up0_ref.at[pl.ds(off + k * 64, 64)])

        @pl.when(c_ax == 1)
        def _():
            pltpu.sync_copy(zrows, up1_ref.at[pl.ds(off + k * 64, 64)])
        return carry
    lax.fori_loop(0, tslice // 64, dr_body, 0)


def kernel(x, y, pos, edge_index, W_unary, g_unary, b_unary, W_pair, g_pair,
           b_pair, c, W_mlp, g_mlp, b_mlp, W_fuse, g_fuse, b_fuse):
    n, d = x.shape
    e = edge_index.shape[1]
    h = W_unary.shape[1]
    out_d = W_mlp.shape[1]
    assert e % CH == 0
    np2 = _cdiv(n, 1024) * 1024
    rrows = 1024
    nb = np2 // rrows
    nch = e // CH
    SDS = jax.ShapeDtypeStruct

    xp = jnp.pad(x, ((0, np2 - n), (0, 0)))
    yp = jnp.pad(y, ((0, np2 - n), (0, 0)))
    nchp = _cdiv(nch, 2 * NWK) * (2 * NWK)
    epad = nchp * CH - e
    i2 = jnp.concatenate(
        [edge_index[0], jnp.full((epad,), np2 - 1, i32)]).reshape(nchp, CH)
    j2 = jnp.concatenate(
        [edge_index[1], jnp.zeros((epad,), i32)]).reshape(nchp, CH)
    wf_h = W_fuse[:out_d]
    wf_y = W_fuse[out_d:]

    bc, cb = pl.pallas_call(
        _prep_w_body,
        out_shape=[SDS((h, h), f32), SDS((h, h), f32)],
    )(c)

    rowblk = lambda bdim: pl.BlockSpec((rrows, bdim), lambda b: (b, 0))
    full2 = lambda s0, s1: pl.BlockSpec((s0, s1), lambda b: (0, 0))
    full1 = lambda s0: pl.BlockSpec((s0,), lambda b: (0,))
    statblk = lambda k: pl.BlockSpec((1, 1, k, h), lambda b: (b, 0, 0, 0))
    statall = lambda k: pl.BlockSpec((nb, 1, k, h), lambda b: (0, 0, 0, 0))
    statblk_o = lambda k: pl.BlockSpec((1, 1, k, out_d), lambda b: (b, 0, 0, 0))
    statall_o = lambda k: pl.BlockSpec((nb, 1, k, out_d), lambda b: (0, 0, 0, 0))

    p1, p2, s1 = pl.pallas_call(
        _pre_body,
        grid=(nb,),
        in_specs=[rowblk(d), rowblk(d), full2(d, h), full2(d, h)],
        out_specs=[rowblk(h), rowblk(h), statblk(4)],
        out_shape=[SDS((np2, h), f32), SDS((np2, h), f32),
                   SDS((nb, 1, 4, h), f32)],
    )(xp, yp, W_unary, W_pair)

    s_nodes, zb, h0 = pl.pallas_call(
        functools.partial(_norm_body, n),
        grid=(nb,),
        in_specs=[rowblk(h), rowblk(h), statall(4), full1(h), full1(h),
                  full1(h), full1(h), full2(h, h)],
        out_specs=[rowblk(h), rowblk(h), rowblk(h)],
        out_shape=[SDS((np2, h), f32), SDS((np2, h), f32), SDS((np2, h), f32)],
    )(p1, p2, s1, g_unary, b_unary, g_pair, b_pair, bc)

    mesh = plsc.VectorSubcoreMesh(core_axis_name="c", subcore_axis_name="s",
                                  num_cores=NC, num_subcores=NS)
    sems6 = [pltpu.SemaphoreType.DMA] * 6
    ex2, den0, den1 = pl.kernel(
        functools.partial(_sc_pass1_body, nchp, h),
        out_type=[SDS((nchp, CH), f32), SDS((np2,), f32), SDS((np2,), f32)],
        mesh=mesh,
        compiler_params=pltpu.CompilerParams(use_tc_tiling_on_sc=False),
        scratch_types=[
            pltpu.VMEM((CH,), i32),
            pltpu.VMEM((CH,), i32),
            pltpu.VMEM((CH,), i32),
            pltpu.VMEM((CH,), i32),
            pltpu.VMEM((CH,), f32),
            pltpu.VMEM((CH,), f32),
            pltpu.VMEM((CH, h), f32),
            pltpu.VMEM((CH, h), f32),
            pltpu.VMEM((CH, h), f32),
            pltpu.VMEM((CH, h), f32),
            pltpu.VMEM((np2 // NS,), f32),
        ] + sems6 + [
            pltpu.VMEM_SHARED((np2,), f32),
        ],
    )(s_nodes, i2, j2)
    den0 = den0.reshape(np2, 1)
    den1 = den1.reshape(np2, 1)

    hcur = h0
    for _ in range(2):
        up0, up1 = pl.kernel(
            functools.partial(_sc_pass2_body, nchp, h),
            out_type=[SDS((np2, h), f32), SDS((np2, h), f32)],
            mesh=mesh,
            compiler_params=pltpu.CompilerParams(use_tc_tiling_on_sc=False),
            scratch_types=[
                pltpu.VMEM((CH,), i32),
                pltpu.VMEM((CH,), i32),
                pltpu.VMEM((CH,), f32),
                pltpu.VMEM((CH,), i32),
                pltpu.VMEM((CH,), i32),
                pltpu.VMEM((CH,), f32),
                pltpu.VMEM((CH, h), f32),
                pltpu.VMEM((CH, h), f32),
                pltpu.VMEM((64, h), f32),
            ] + sems6 + [
                pltpu.VMEM_SHARED((np2, h), f32),
            ],
        )(hcur, ex2, i2, j2)

        colblk = pl.BlockSpec((rrows, 1), lambda b: (b, 0))
        hcur = pl.pallas_call(
            functools.partial(_update_body, n, rrows),
            grid=(nb,),
            in_specs=[rowblk(h), rowblk(h), colblk, colblk,
                      rowblk(h), full2(h, h)],
            out_specs=rowblk(h),
            out_shape=SDS((np2, h), f32),
        )(up0, up1, den0, den1, zb, cb)

    g1, s2 = pl.pallas_call(
        _posta_body,
        grid=(nb,),
        in_specs=[rowblk(h), full2(h, out_d)],
        out_specs=[rowblk(out_d), statblk_o(2)],
        out_shape=[SDS((np2, out_d), f32), SDS((nb, 1, 2, out_d), f32)],
    )(hcur, W_mlp)

    g2, s3 = pl.pallas_call(
        functools.partial(_postb_body, n, rrows),
        grid=(nb,),
        in_specs=[rowblk(out_d), statall_o(2), full1(out_d), full1(out_d),
                  rowblk(d), full2(out_d, out_d), full2(d, out_d)],
        out_specs=[rowblk(out_d), statblk_o(2)],
        out_shape=[SDS((np2, out_d), f32), SDS((nb, 1, 2, out_d), f32)],
    )(g1, s2, g_mlp, b_mlp, yp, wf_h, wf_y)

    outp = pl.pallas_call(
        functools.partial(_postc_body, n),
        grid=(nb,),
        in_specs=[rowblk(out_d), statall_o(2), full1(out_d), full1(out_d)],
        out_specs=rowblk(out_d),
        out_shape=SDS((np2, out_d), f32),
    )(g2, s3, g_fuse, b_fuse)

    return outp[:n]
